# baseline (speedup 1.0000x reference)
import functools

import jax
import jax.numpy as jnp
from jax.experimental import pallas as pl
from jax.experimental.pallas import tpu as pltpu

S = 2048
D = 1024
FF = 4096
E = 8
BS = 1024
FFC = 1024
NS = S // BS


def _router_body(x_ref, wr_ref, probs_ref, ti_ref, tw_ref, comb_ref):
    x = x_ref[...]
    wr = wr_ref[...]
    logits = jax.lax.dot_general(
        x, wr, (((1,), (1,)), ((), ())), preferred_element_type=jnp.float32)
    m = jnp.max(logits, axis=1, keepdims=True)
    ex = jnp.exp(logits - m)
    probs = ex / jnp.sum(ex, axis=1, keepdims=True)
    probs_ref[...] = probs

    lane = jax.lax.broadcasted_iota(jnp.int32, (S, E), 1)
    m0 = jnp.max(probs, axis=1, keepdims=True)
    i0 = jnp.min(jnp.where(probs == m0, lane, E), axis=1, keepdims=True)
    masked = jnp.where(lane == i0, -jnp.inf, probs)
    m1 = jnp.max(masked, axis=1, keepdims=True)
    i1 = jnp.min(jnp.where(masked == m1, lane, E), axis=1, keepdims=True)

    denom = m0 + m1
    w0 = m0 / denom
    w1 = m1 / denom
    ti_ref[...] = jnp.concatenate([i0, i1], axis=1)
    tw_ref[...] = jnp.concatenate([w0, w1], axis=1)
    comb_ref[...] = jnp.where(lane == i0, w0, 0.0) + jnp.where(lane == i1, w1, 0.0)


def _router(x2d, wr):
    return pl.pallas_call(
        _router_body,
        out_shape=(
            jax.ShapeDtypeStruct((S, E), jnp.float32),
            jax.ShapeDtypeStruct((S, 2), jnp.int32),
            jax.ShapeDtypeStruct((S, 2), jnp.float32),
            jax.ShapeDtypeStruct((S, E), jnp.float32),
        ),
    )(x2d, wr)


def _dense_body(x_ref, w1_ref, w2_ref, c_ref, o_ref):
    e = pl.program_id(1)
    xb = x_ref[...].astype(jnp.bfloat16)
    lane = jax.lax.broadcasted_iota(jnp.int32, (BS, E), 1)
    ce = jnp.sum(jnp.where(lane == e, c_ref[...], 0.0), axis=1, keepdims=True)

    part = jnp.zeros((BS, D), jnp.float32)
    for f in range(FF // FFC):
        w1c = w1_ref[0, f * FFC:(f + 1) * FFC, :]
        h = jax.lax.dot_general(
            xb, w1c, (((1,), (1,)), ((), ())), preferred_element_type=jnp.float32)
        h = h * jax.nn.sigmoid(h)
        w2c = w2_ref[0, :, f * FFC:(f + 1) * FFC]
        part = part + jax.lax.dot_general(
            h.astype(jnp.bfloat16), w2c, (((1,), (1,)), ((), ())),
            preferred_element_type=jnp.float32)

    contrib = ce * part

    @pl.when(e == 0)
    def _():
        o_ref[...] = contrib

    @pl.when(e != 0)
    def _():
        o_ref[...] += contrib


def _dense_moe(x2d, w1b, w2b, comb):
    return pl.pallas_call(
        _dense_body,
        grid=(NS, E),
        in_specs=[
            pl.BlockSpec((BS, D), lambda s, e: (s, 0)),
            pl.BlockSpec((1, FF, D), lambda s, e: (e, 0, 0)),
            pl.BlockSpec((1, D, FF), lambda s, e: (e, 0, 0)),
            pl.BlockSpec((BS, E), lambda s, e: (s, 0)),
        ],
        out_specs=pl.BlockSpec((BS, D), lambda s, e: (s, 0)),
        out_shape=jax.ShapeDtypeStruct((S, D), jnp.float32),
    )(x2d, w1b, w2b, comb)


@jax.jit
def kernel(x, Wr, w1, w2):
    x2d = x.reshape(S, D)
    probs, ti, tw, comb = _router(x2d, Wr)
    w1b = w1.astype(jnp.bfloat16)
    w2b = w2.astype(jnp.bfloat16)
    out = _dense_moe(x2d, w1b, w2b, comb)
    return (out.reshape(1, S, D), probs.reshape(1, S, E),
            ti.reshape(1, S, 2), tw.reshape(1, S, 2))


# speedup vs baseline: 1.4881x; 1.4881x over previous
import functools

import jax
import jax.numpy as jnp
from jax import lax
from jax.experimental import pallas as pl
from jax.experimental.pallas import tpu as pltpu
from jax.experimental.pallas import tpu_sc as plsc

S = 2048
D = 1024
FF = 4096
E = 8
K = 2
A = S * K
TB = 256
P = A + E * TB
NB = P // TB
FFC = 1024
CH = 64
NW = 32


def _router_body(x_ref, wr_ref, probs_ref, ti_ref, tw_ref, pos_ref, be_ref):
    x = x_ref[...]
    wr = wr_ref[...]
    logits = lax.dot_general(
        x, wr, (((1,), (1,)), ((), ())), preferred_element_type=jnp.float32)
    m = jnp.max(logits, axis=1, keepdims=True)
    ex = jnp.exp(logits - m)
    probs = ex / jnp.sum(ex, axis=1, keepdims=True)
    probs_ref[...] = probs

    lane = lax.broadcasted_iota(jnp.int32, (S, E), 1)
    m0 = jnp.max(probs, axis=1, keepdims=True)
    i0 = jnp.min(jnp.where(probs == m0, lane, E), axis=1, keepdims=True)
    masked = jnp.where(lane == i0, -jnp.inf, probs)
    m1 = jnp.max(masked, axis=1, keepdims=True)
    i1 = jnp.min(jnp.where(masked == m1, lane, E), axis=1, keepdims=True)

    denom = m0 + m1
    ti_ref[...] = jnp.concatenate([i0, i1], axis=1)
    tw_ref[...] = jnp.concatenate([m0 / denom, m1 / denom], axis=1)

    h0 = (lane == i0).astype(jnp.bfloat16)
    h1 = (lane == i1).astype(jnp.bfloat16)
    mm = jnp.concatenate([h0, h1], axis=0)

    counts = jnp.sum(mm.astype(jnp.float32), axis=0, keepdims=True)
    nb = jnp.floor((counts + (TB - 1)) * (1.0 / TB))

    e_sub = lax.broadcasted_iota(jnp.int32, (E, E), 0)
    e_lane = lax.broadcasted_iota(jnp.int32, (E, E), 1)
    upper = (e_sub < e_lane).astype(jnp.bfloat16)
    off_row = lax.dot_general(
        nb.astype(jnp.bfloat16), upper, (((1,), (0,)), ((), ())),
        preferred_element_type=jnp.float32) * TB

    chunk = 512
    r_sub = lax.broadcasted_iota(jnp.int32, (chunk, chunk), 0)
    r_lane = lax.broadcasted_iota(jnp.int32, (chunk, chunk), 1)
    tril = (r_lane < r_sub).astype(jnp.bfloat16)
    carry = jnp.zeros((1, E), jnp.float32)
    pos_chunks = []
    for c in range(A // chunk):
        mc = mm[c * chunk:(c + 1) * chunk, :]
        cc = lax.dot_general(
            tril, mc, (((1,), (0,)), ((), ())),
            preferred_element_type=jnp.float32) + carry
        mcf = mc.astype(jnp.float32)
        posc = jnp.sum((cc + off_row) * mcf, axis=1, keepdims=True)
        pos_chunks.append(posc)
        carry = carry + jnp.sum(mcf, axis=0, keepdims=True)
    pos_ref[...] = jnp.concatenate(pos_chunks, axis=0).astype(jnp.int32)

    counts_col = lax.dot_general(
        mm, jnp.ones((A, 1), jnp.bfloat16), (((0,), (0,)), ((), ())),
        preferred_element_type=jnp.float32)
    nb_col = jnp.floor((counts_col + (TB - 1)) * (1.0 / TB))
    lower = (e_lane < e_sub).astype(jnp.bfloat16)
    boff_col = lax.dot_general(
        lower, nb_col, (((1,), (0,)), ((), ())),
        preferred_element_type=jnp.float32)
    b_lane = lax.broadcasted_iota(jnp.int32, (E, 128), 1)
    bmask = (boff_col <= b_lane.astype(jnp.float32)).astype(jnp.float32)
    be = jnp.sum(bmask, axis=0, keepdims=True) - 1.0
    used = jnp.sum(nb)
    one_lane = lax.broadcasted_iota(jnp.int32, (1, 128), 1)
    be_ref[...] = jnp.where(one_lane == NB, used, be).astype(jnp.int32)


def _router(x2d, wr):
    return pl.pallas_call(
        _router_body,
        out_shape=(
            jax.ShapeDtypeStruct((S, E), jnp.float32),
            jax.ShapeDtypeStruct((S, K), jnp.int32),
            jax.ShapeDtypeStruct((S, K), jnp.float32),
            jax.ShapeDtypeStruct((A, 1), jnp.int32),
            jax.ShapeDtypeStruct((1, 128), jnp.int32),
        ),
    )(x2d, wr)


def _sc_dispatch(x2d, tid, pos):
    mesh = plsc.VectorSubcoreMesh(core_axis_name="c", subcore_axis_name="s")

    @functools.partial(
        pl.kernel, mesh=mesh,
        out_type=jax.ShapeDtypeStruct((P, D), jnp.float32),
        scratch_types=[
            pltpu.VMEM((CH,), jnp.int32),
            pltpu.VMEM((CH,), jnp.int32),
            pltpu.VMEM((CH, D), jnp.float32),
            pltpu.SemaphoreType.DMA,
        ],
    )
    def disp(x_hbm, tid_hbm, pos_hbm, xg_hbm, tid_v, pos_v, rows_v, sem):
        wid = lax.axis_index("s") * 2 + lax.axis_index("c")
        for j in range(A // (NW * CH)):
            base = wid * (A // NW) + j * CH
            pltpu.sync_copy(tid_hbm.at[pl.ds(base, CH)], tid_v)
            pltpu.sync_copy(pos_hbm.at[pl.ds(base, CH)], pos_v)
            pltpu.async_copy(x_hbm.at[tid_v], rows_v, sem).wait()
            pltpu.async_copy(rows_v, xg_hbm.at[pos_v], sem).wait()

    return disp(x2d, tid, pos)


def _sc_collect(og, pos):
    mesh = plsc.VectorSubcoreMesh(core_axis_name="c", subcore_axis_name="s")

    @functools.partial(
        pl.kernel, mesh=mesh,
        out_type=jax.ShapeDtypeStruct((A, D), jnp.float32),
        scratch_types=[
            pltpu.VMEM((CH,), jnp.int32),
            pltpu.VMEM((CH, D), jnp.float32),
            pltpu.SemaphoreType.DMA,
        ],
    )
    def coll(og_hbm, pos_hbm, gall_hbm, pos_v, rows_v, sem):
        wid = lax.axis_index("s") * 2 + lax.axis_index("c")
        for j in range(A // (NW * CH)):
            base = wid * (A // NW) + j * CH
            pltpu.sync_copy(pos_hbm.at[pl.ds(base, CH)], pos_v)
            pltpu.async_copy(og_hbm.at[pos_v], rows_v, sem).wait()
            pltpu.sync_copy(rows_v, gall_hbm.at[pl.ds(base, CH)])

    return coll(og, pos)


def _ffn_body(sc_ref, xg_ref, w1_ref, w2_ref, o_ref):
    b = pl.program_id(0)

    @pl.when(b < sc_ref[NB])
    def _():
        xb = xg_ref[...].astype(jnp.bfloat16)
        acc = jnp.zeros((TB, D), jnp.float32)
        for f in range(FF // FFC):
            w1c = w1_ref[0, f * FFC:(f + 1) * FFC, :]
            h = lax.dot_general(
                xb, w1c, (((1,), (1,)), ((), ())),
                preferred_element_type=jnp.float32)
            h = h * jax.nn.sigmoid(h)
            w2c = w2_ref[0, :, f * FFC:(f + 1) * FFC]
            acc = acc + lax.dot_general(
                h.astype(jnp.bfloat16), w2c, (((1,), (1,)), ((), ())),
                preferred_element_type=jnp.float32)
        o_ref[...] = acc


def _grouped_ffn(scalars, xg, w1b, w2b):
    grid_spec = pltpu.PrefetchScalarGridSpec(
        num_scalar_prefetch=1,
        grid=(NB,),
        in_specs=[
            pl.BlockSpec((TB, D), lambda b, sc: (b, 0)),
            pl.BlockSpec((1, FF, D), lambda b, sc: (sc[b], 0, 0)),
            pl.BlockSpec((1, D, FF), lambda b, sc: (sc[b], 0, 0)),
        ],
        out_specs=pl.BlockSpec((TB, D), lambda b, sc: (b, 0)),
    )
    return pl.pallas_call(
        _ffn_body,
        grid_spec=grid_spec,
        out_shape=jax.ShapeDtypeStruct((P, D), jnp.float32),
    )(scalars, xg, w1b, w2b)


def _combine_body(g_ref, tw_ref, o_ref):
    g0 = g_ref[0:S, :]
    g1 = g_ref[S:A, :]
    o_ref[...] = tw_ref[:, 0:1] * g0 + tw_ref[:, 1:2] * g1


def _combine(g_all, tw):
    return pl.pallas_call(
        _combine_body,
        out_shape=jax.ShapeDtypeStruct((S, D), jnp.float32),
    )(g_all, tw)


@jax.jit
def kernel(x, Wr, w1, w2):
    x2d = x.reshape(S, D)
    probs, ti, tw, posf, bev = _router(x2d, Wr)
    pos = posf.reshape(A)
    tid = jnp.tile(jnp.arange(S, dtype=jnp.int32), K)
    xg = _sc_dispatch(x2d, tid, pos)
    w1b = w1.astype(jnp.bfloat16)
    w2b = w2.astype(jnp.bfloat16)
    scalars = bev[0, :NB + 1]
    og = _grouped_ffn(scalars, xg, w1b, w2b)
    g_all = _sc_collect(og, pos)
    out = _combine(g_all, tw)
    return (out.reshape(1, S, D), probs.reshape(1, S, E),
            ti.reshape(1, S, 2), tw.reshape(1, S, 2))


# speedup vs baseline: 1.7797x; 1.1960x over previous
import functools

import jax
import jax.numpy as jnp
from jax import lax
from jax.experimental import pallas as pl
from jax.experimental.pallas import tpu as pltpu
from jax.experimental.pallas import tpu_sc as plsc

S = 2048
D = 1024
FF = 4096
E = 8
K = 2
A = S * K
TB = 256
P = A + E * TB
NB = P // TB
FFC = 1024
CH = 64
NW = 32


def _router_body(x_ref, wr_ref, probs_ref, ti_ref, tw_ref, pos_ref, be_ref):
    x = x_ref[...]
    wr = wr_ref[...]
    logits = lax.dot_general(
        x, wr, (((1,), (1,)), ((), ())), preferred_element_type=jnp.float32)
    m = jnp.max(logits, axis=1, keepdims=True)
    ex = jnp.exp(logits - m)
    probs = ex / jnp.sum(ex, axis=1, keepdims=True)
    probs_ref[...] = probs

    lane = lax.broadcasted_iota(jnp.int32, (S, E), 1)
    m0 = jnp.max(probs, axis=1, keepdims=True)
    i0 = jnp.min(jnp.where(probs == m0, lane, E), axis=1, keepdims=True)
    masked = jnp.where(lane == i0, -jnp.inf, probs)
    m1 = jnp.max(masked, axis=1, keepdims=True)
    i1 = jnp.min(jnp.where(masked == m1, lane, E), axis=1, keepdims=True)

    denom = m0 + m1
    ti_ref[...] = jnp.concatenate([i0, i1], axis=1)
    tw_ref[...] = jnp.concatenate([m0 / denom, m1 / denom], axis=1)

    h0 = (lane == i0).astype(jnp.bfloat16)
    h1 = (lane == i1).astype(jnp.bfloat16)
    mm = jnp.concatenate([h0, h1], axis=0)

    counts = jnp.sum(mm.astype(jnp.float32), axis=0, keepdims=True)
    nb = jnp.floor((counts + (TB - 1)) * (1.0 / TB))

    e_sub = lax.broadcasted_iota(jnp.int32, (E, E), 0)
    e_lane = lax.broadcasted_iota(jnp.int32, (E, E), 1)
    upper = (e_sub < e_lane).astype(jnp.bfloat16)
    off_row = lax.dot_general(
        nb.astype(jnp.bfloat16), upper, (((1,), (0,)), ((), ())),
        preferred_element_type=jnp.float32) * TB

    chunk = 512
    r_sub = lax.broadcasted_iota(jnp.int32, (chunk, chunk), 0)
    r_lane = lax.broadcasted_iota(jnp.int32, (chunk, chunk), 1)
    tril = (r_lane < r_sub).astype(jnp.bfloat16)
    carry = jnp.zeros((1, E), jnp.float32)
    pos_chunks = []
    for c in range(A // chunk):
        mc = mm[c * chunk:(c + 1) * chunk, :]
        cc = lax.dot_general(
            tril, mc, (((1,), (0,)), ((), ())),
            preferred_element_type=jnp.float32) + carry
        mcf = mc.astype(jnp.float32)
        posc = jnp.sum((cc + off_row) * mcf, axis=1, keepdims=True)
        pos_chunks.append(posc)
        carry = carry + jnp.sum(mcf, axis=0, keepdims=True)
    pos_ref[...] = jnp.concatenate(pos_chunks, axis=0).astype(jnp.int32)

    counts_col = lax.dot_general(
        mm, jnp.ones((A, 1), jnp.bfloat16), (((0,), (0,)), ((), ())),
        preferred_element_type=jnp.float32)
    nb_col = jnp.floor((counts_col + (TB - 1)) * (1.0 / TB))
    lower = (e_lane < e_sub).astype(jnp.bfloat16)
    boff_col = lax.dot_general(
        lower, nb_col, (((1,), (0,)), ((), ())),
        preferred_element_type=jnp.float32)
    b_lane = lax.broadcasted_iota(jnp.int32, (E, 128), 1)
    bmask = (boff_col <= b_lane.astype(jnp.float32)).astype(jnp.float32)
    be = jnp.sum(bmask, axis=0, keepdims=True) - 1.0
    used = jnp.sum(nb)
    one_lane = lax.broadcasted_iota(jnp.int32, (1, 128), 1)
    be_ref[...] = jnp.where(one_lane == NB, used, be).astype(jnp.int32)


def _router(x2d, wr):
    return pl.pallas_call(
        _router_body,
        out_shape=(
            jax.ShapeDtypeStruct((S, E), jnp.float32),
            jax.ShapeDtypeStruct((S, K), jnp.int32),
            jax.ShapeDtypeStruct((S, K), jnp.float32),
            jax.ShapeDtypeStruct((A, 1), jnp.int32),
            jax.ShapeDtypeStruct((1, 128), jnp.int32),
        ),
    )(x2d, wr)


def _sc_dispatch(x2d, tid, pos):
    mesh = plsc.VectorSubcoreMesh(core_axis_name="c", subcore_axis_name="s")

    @functools.partial(
        pl.kernel, mesh=mesh,
        out_type=jax.ShapeDtypeStruct((P, D), jnp.float32),
        scratch_types=[
            pltpu.VMEM((CH,), jnp.int32),
            pltpu.VMEM((CH,), jnp.int32),
            pltpu.VMEM((CH, D), jnp.float32),
            pltpu.SemaphoreType.DMA,
        ],
    )
    def disp(x_hbm, tid_hbm, pos_hbm, xg_hbm, tid_v, pos_v, rows_v, sem):
        wid = lax.axis_index("s") * 2 + lax.axis_index("c")
        for j in range(A // (NW * CH)):
            base = wid * (A // NW) + j * CH
            pltpu.sync_copy(tid_hbm.at[pl.ds(base, CH)], tid_v)
            pltpu.sync_copy(pos_hbm.at[pl.ds(base, CH)], pos_v)
            pltpu.async_copy(x_hbm.at[tid_v], rows_v, sem).wait()
            pltpu.async_copy(rows_v, xg_hbm.at[pos_v], sem).wait()

    return disp(x2d, tid, pos)


def _sc_collect(og, pos):
    mesh = plsc.VectorSubcoreMesh(core_axis_name="c", subcore_axis_name="s")

    @functools.partial(
        pl.kernel, mesh=mesh,
        out_type=jax.ShapeDtypeStruct((A, D), jnp.float32),
        scratch_types=[
            pltpu.VMEM((CH,), jnp.int32),
            pltpu.VMEM((CH, D), jnp.float32),
            pltpu.SemaphoreType.DMA,
        ],
    )
    def coll(og_hbm, pos_hbm, gall_hbm, pos_v, rows_v, sem):
        wid = lax.axis_index("s") * 2 + lax.axis_index("c")
        for j in range(A // (NW * CH)):
            base = wid * (A // NW) + j * CH
            pltpu.sync_copy(pos_hbm.at[pl.ds(base, CH)], pos_v)
            pltpu.async_copy(og_hbm.at[pos_v], rows_v, sem).wait()
            pltpu.sync_copy(rows_v, gall_hbm.at[pl.ds(base, CH)])

    return coll(og, pos)


FFH = FF // 2


def _ffn_acc(xg_ref, w1_ref, w2_ref):
    xb = xg_ref[...].astype(jnp.bfloat16)
    acc = jnp.zeros((TB, D), jnp.float32)
    for fc in range(FFH // FFC):
        w1c = w1_ref[0, fc * FFC:(fc + 1) * FFC, :].astype(jnp.bfloat16)
        h = lax.dot_general(
            xb, w1c, (((1,), (1,)), ((), ())),
            preferred_element_type=jnp.float32)
        h = h * jax.nn.sigmoid(h)
        w2c = w2_ref[0, :, fc * FFC:(fc + 1) * FFC].astype(jnp.bfloat16)
        acc = acc + lax.dot_general(
            h.astype(jnp.bfloat16), w2c, (((1,), (1,)), ((), ())),
            preferred_element_type=jnp.float32)
    return acc


def _ffn_body_a(sc_ref, xg_ref, w1_ref, w2_ref, o_ref):
    @pl.when(pl.program_id(0) < sc_ref[NB])
    def _():
        o_ref[...] = _ffn_acc(xg_ref, w1_ref, w2_ref)


def _ffn_body_b(sc_ref, xg_ref, w1_ref, w2_ref, prev_ref, o_ref):
    @pl.when(pl.program_id(0) < sc_ref[NB])
    def _():
        o_ref[...] = prev_ref[...] + _ffn_acc(xg_ref, w1_ref, w2_ref)


def _grouped_ffn(scalars, xg, w1, w2):
    spec_a = pltpu.PrefetchScalarGridSpec(
        num_scalar_prefetch=1,
        grid=(NB,),
        in_specs=[
            pl.BlockSpec((TB, D), lambda b, sc: (b, 0)),
            pl.BlockSpec((1, FFH, D), lambda b, sc: (sc[b], 0, 0)),
            pl.BlockSpec((1, D, FFH), lambda b, sc: (sc[b], 0, 0)),
        ],
        out_specs=pl.BlockSpec((TB, D), lambda b, sc: (b, 0)),
    )
    og0 = pl.pallas_call(
        _ffn_body_a,
        grid_spec=spec_a,
        out_shape=jax.ShapeDtypeStruct((P, D), jnp.float32),
    )(scalars, xg, w1, w2)
    spec_b = pltpu.PrefetchScalarGridSpec(
        num_scalar_prefetch=1,
        grid=(NB,),
        in_specs=[
            pl.BlockSpec((TB, D), lambda b, sc: (b, 0)),
            pl.BlockSpec((1, FFH, D), lambda b, sc: (sc[b], 1, 0)),
            pl.BlockSpec((1, D, FFH), lambda b, sc: (sc[b], 0, 1)),
            pl.BlockSpec((TB, D), lambda b, sc: (b, 0)),
        ],
        out_specs=pl.BlockSpec((TB, D), lambda b, sc: (b, 0)),
    )
    return pl.pallas_call(
        _ffn_body_b,
        grid_spec=spec_b,
        out_shape=jax.ShapeDtypeStruct((P, D), jnp.float32),
        input_output_aliases={4: 0},
    )(scalars, xg, w1, w2, og0)


def _combine_body(g_ref, tw_ref, o_ref):
    g0 = g_ref[0:S, :]
    g1 = g_ref[S:A, :]
    o_ref[...] = tw_ref[:, 0:1] * g0 + tw_ref[:, 1:2] * g1


def _combine(g_all, tw):
    return pl.pallas_call(
        _combine_body,
        out_shape=jax.ShapeDtypeStruct((S, D), jnp.float32),
    )(g_all, tw)


@jax.jit
def kernel(x, Wr, w1, w2):
    x2d = x.reshape(S, D)
    probs, ti, tw, posf, bev = _router(x2d, Wr)
    pos = posf.reshape(A)
    tid = jnp.tile(jnp.arange(S, dtype=jnp.int32), K)
    xg = _sc_dispatch(x2d, tid, pos)
    scalars = bev[0, :NB + 1]
    og = _grouped_ffn(scalars, xg, w1, w2)
    g_all = _sc_collect(og, pos)
    out = _combine(g_all, tw)
    return (out.reshape(1, S, D), probs.reshape(1, S, E),
            ti.reshape(1, S, 2), tw.reshape(1, S, 2))


# speedup vs baseline: 1.9115x; 1.0741x over previous
import functools

import jax
import jax.numpy as jnp
from jax import lax
from jax.experimental import pallas as pl
from jax.experimental.pallas import tpu as pltpu
from jax.experimental.pallas import tpu_sc as plsc

S = 2048
D = 1024
FF = 4096
E = 8
K = 2
A = S * K
TB = 512
P = A + E * TB
NB = P // TB
FFC = 1024
CH = 64
NW = 32


def _router_body(x_ref, wr_ref, probs_ref, ti_ref, tw_ref, pos_ref, be_ref):
    x = x_ref[...]
    wr = wr_ref[...]
    logits = lax.dot_general(
        x, wr, (((1,), (1,)), ((), ())), preferred_element_type=jnp.float32)
    m = jnp.max(logits, axis=1, keepdims=True)
    ex = jnp.exp(logits - m)
    probs = ex / jnp.sum(ex, axis=1, keepdims=True)
    probs_ref[...] = probs

    lane = lax.broadcasted_iota(jnp.int32, (S, E), 1)
    m0 = jnp.max(probs, axis=1, keepdims=True)
    i0 = jnp.min(jnp.where(probs == m0, lane, E), axis=1, keepdims=True)
    masked = jnp.where(lane == i0, -jnp.inf, probs)
    m1 = jnp.max(masked, axis=1, keepdims=True)
    i1 = jnp.min(jnp.where(masked == m1, lane, E), axis=1, keepdims=True)

    denom = m0 + m1
    ti_ref[...] = jnp.concatenate([i0, i1], axis=1)
    tw_ref[...] = jnp.concatenate([m0 / denom, m1 / denom], axis=1)

    h0 = (lane == i0).astype(jnp.bfloat16)
    h1 = (lane == i1).astype(jnp.bfloat16)
    mm = jnp.concatenate([h0, h1], axis=0)

    counts = jnp.sum(mm.astype(jnp.float32), axis=0, keepdims=True)
    nb = jnp.floor((counts + (TB - 1)) * (1.0 / TB))

    e_sub = lax.broadcasted_iota(jnp.int32, (E, E), 0)
    e_lane = lax.broadcasted_iota(jnp.int32, (E, E), 1)
    upper = (e_sub < e_lane).astype(jnp.bfloat16)
    off_row = lax.dot_general(
        nb.astype(jnp.bfloat16), upper, (((1,), (0,)), ((), ())),
        preferred_element_type=jnp.float32) * TB

    chunk = 512
    r_sub = lax.broadcasted_iota(jnp.int32, (chunk, chunk), 0)
    r_lane = lax.broadcasted_iota(jnp.int32, (chunk, chunk), 1)
    tril = (r_lane < r_sub).astype(jnp.bfloat16)
    carry = jnp.zeros((1, E), jnp.float32)
    pos_chunks = []
    for c in range(A // chunk):
        mc = mm[c * chunk:(c + 1) * chunk, :]
        cc = lax.dot_general(
            tril, mc, (((1,), (0,)), ((), ())),
            preferred_element_type=jnp.float32) + carry
        mcf = mc.astype(jnp.float32)
        posc = jnp.sum((cc + off_row) * mcf, axis=1, keepdims=True)
        pos_chunks.append(posc)
        carry = carry + jnp.sum(mcf, axis=0, keepdims=True)
    pos_ref[...] = jnp.concatenate(pos_chunks, axis=0).astype(jnp.int32)

    counts_col = lax.dot_general(
        mm, jnp.ones((A, 1), jnp.bfloat16), (((0,), (0,)), ((), ())),
        preferred_element_type=jnp.float32)
    nb_col = jnp.floor((counts_col + (TB - 1)) * (1.0 / TB))
    lower = (e_lane < e_sub).astype(jnp.bfloat16)
    boff_col = lax.dot_general(
        lower, nb_col, (((1,), (0,)), ((), ())),
        preferred_element_type=jnp.float32)
    b_lane = lax.broadcasted_iota(jnp.int32, (E, 128), 1)
    bmask = (boff_col <= b_lane.astype(jnp.float32)).astype(jnp.float32)
    be = jnp.sum(bmask, axis=0, keepdims=True) - 1.0
    used = jnp.sum(nb)
    one_lane = lax.broadcasted_iota(jnp.int32, (1, 128), 1)
    be_ref[...] = jnp.where(one_lane == NB, used, be).astype(jnp.int32)


def _router(x2d, wr):
    return pl.pallas_call(
        _router_body,
        out_shape=(
            jax.ShapeDtypeStruct((S, E), jnp.float32),
            jax.ShapeDtypeStruct((S, K), jnp.int32),
            jax.ShapeDtypeStruct((S, K), jnp.float32),
            jax.ShapeDtypeStruct((A, 1), jnp.int32),
            jax.ShapeDtypeStruct((1, 128), jnp.int32),
        ),
    )(x2d, wr)


def _sc_dispatch(x2d, tid, pos):
    mesh = plsc.VectorSubcoreMesh(core_axis_name="c", subcore_axis_name="s")

    @functools.partial(
        pl.kernel, mesh=mesh,
        out_type=jax.ShapeDtypeStruct((P, D), jnp.float32),
        scratch_types=[
            pltpu.VMEM((CH,), jnp.int32),
            pltpu.VMEM((CH,), jnp.int32),
            pltpu.VMEM((CH, D), jnp.float32),
            pltpu.SemaphoreType.DMA,
        ],
    )
    def disp(x_hbm, tid_hbm, pos_hbm, xg_hbm, tid_v, pos_v, rows_v, sem):
        wid = lax.axis_index("s") * 2 + lax.axis_index("c")
        for j in range(A // (NW * CH)):
            base = wid * (A // NW) + j * CH
            pltpu.sync_copy(tid_hbm.at[pl.ds(base, CH)], tid_v)
            pltpu.sync_copy(pos_hbm.at[pl.ds(base, CH)], pos_v)
            pltpu.async_copy(x_hbm.at[tid_v], rows_v, sem).wait()
            pltpu.async_copy(rows_v, xg_hbm.at[pos_v], sem).wait()

    return disp(x2d, tid, pos)


def _sc_collect(og, pos):
    mesh = plsc.VectorSubcoreMesh(core_axis_name="c", subcore_axis_name="s")

    @functools.partial(
        pl.kernel, mesh=mesh,
        out_type=jax.ShapeDtypeStruct((A, D), jnp.float32),
        scratch_types=[
            pltpu.VMEM((CH,), jnp.int32),
            pltpu.VMEM((CH, D), jnp.float32),
            pltpu.SemaphoreType.DMA,
        ],
    )
    def coll(og_hbm, pos_hbm, gall_hbm, pos_v, rows_v, sem):
        wid = lax.axis_index("s") * 2 + lax.axis_index("c")
        for j in range(A // (NW * CH)):
            base = wid * (A // NW) + j * CH
            pltpu.sync_copy(pos_hbm.at[pl.ds(base, CH)], pos_v)
            pltpu.async_copy(og_hbm.at[pos_v], rows_v, sem).wait()
            pltpu.sync_copy(rows_v, gall_hbm.at[pl.ds(base, CH)])

    return coll(og, pos)


FFH = FF // 2


def _ffn_acc(xg_ref, w1_ref, w2_ref):
    xb = xg_ref[...].astype(jnp.bfloat16)
    acc = jnp.zeros((TB, D), jnp.float32)
    for fc in range(FFH // FFC):
        w1c = w1_ref[0, fc * FFC:(fc + 1) * FFC, :].astype(jnp.bfloat16)
        h = lax.dot_general(
            xb, w1c, (((1,), (1,)), ((), ())),
            preferred_element_type=jnp.float32)
        h = h * jax.nn.sigmoid(h)
        w2c = w2_ref[0, :, fc * FFC:(fc + 1) * FFC].astype(jnp.bfloat16)
        acc = acc + lax.dot_general(
            h.astype(jnp.bfloat16), w2c, (((1,), (1,)), ((), ())),
            preferred_element_type=jnp.float32)
    return acc


def _ffn_body_a(sc_ref, xg_ref, w1_ref, w2_ref, o_ref):
    @pl.when(pl.program_id(0) < sc_ref[NB])
    def _():
        o_ref[...] = _ffn_acc(xg_ref, w1_ref, w2_ref)


def _ffn_body_b(sc_ref, xg_ref, w1_ref, w2_ref, prev_ref, o_ref):
    @pl.when(pl.program_id(0) < sc_ref[NB])
    def _():
        o_ref[...] = prev_ref[...] + _ffn_acc(xg_ref, w1_ref, w2_ref)


def _grouped_ffn(scalars, xg, w1, w2):
    spec_a = pltpu.PrefetchScalarGridSpec(
        num_scalar_prefetch=1,
        grid=(NB,),
        in_specs=[
            pl.BlockSpec((TB, D), lambda b, sc: (b, 0)),
            pl.BlockSpec((1, FFH, D), lambda b, sc: (sc[b], 0, 0)),
            pl.BlockSpec((1, D, FFH), lambda b, sc: (sc[b], 0, 0)),
        ],
        out_specs=pl.BlockSpec((TB, D), lambda b, sc: (b, 0)),
    )
    og0 = pl.pallas_call(
        _ffn_body_a,
        grid_spec=spec_a,
        out_shape=jax.ShapeDtypeStruct((P, D), jnp.float32),
    )(scalars, xg, w1, w2)
    spec_b = pltpu.PrefetchScalarGridSpec(
        num_scalar_prefetch=1,
        grid=(NB,),
        in_specs=[
            pl.BlockSpec((TB, D), lambda b, sc: (b, 0)),
            pl.BlockSpec((1, FFH, D), lambda b, sc: (sc[b], 1, 0)),
            pl.BlockSpec((1, D, FFH), lambda b, sc: (sc[b], 0, 1)),
            pl.BlockSpec((TB, D), lambda b, sc: (b, 0)),
        ],
        out_specs=pl.BlockSpec((TB, D), lambda b, sc: (b, 0)),
    )
    return pl.pallas_call(
        _ffn_body_b,
        grid_spec=spec_b,
        out_shape=jax.ShapeDtypeStruct((P, D), jnp.float32),
        input_output_aliases={4: 0},
    )(scalars, xg, w1, w2, og0)


def _combine_body(g_ref, tw_ref, o_ref):
    g0 = g_ref[0:S, :]
    g1 = g_ref[S:A, :]
    o_ref[...] = tw_ref[:, 0:1] * g0 + tw_ref[:, 1:2] * g1


def _combine(g_all, tw):
    return pl.pallas_call(
        _combine_body,
        out_shape=jax.ShapeDtypeStruct((S, D), jnp.float32),
    )(g_all, tw)


@jax.jit
def kernel(x, Wr, w1, w2):
    x2d = x.reshape(S, D)
    probs, ti, tw, posf, bev = _router(x2d, Wr)
    pos = posf.reshape(A)
    tid = jnp.tile(jnp.arange(S, dtype=jnp.int32), K)
    xg = _sc_dispatch(x2d, tid, pos)
    scalars = bev[0, :NB + 1]
    og = _grouped_ffn(scalars, xg, w1, w2)
    g_all = _sc_collect(og, pos)
    out = _combine(g_all, tw)
    return (out.reshape(1, S, D), probs.reshape(1, S, E),
            ti.reshape(1, S, 2), tw.reshape(1, S, 2))


# speedup vs baseline: 1.9677x; 1.0294x over previous
import functools

import jax
import jax.numpy as jnp
from jax import lax
from jax.experimental import pallas as pl
from jax.experimental.pallas import tpu as pltpu
from jax.experimental.pallas import tpu_sc as plsc

S = 2048
D = 1024
FF = 4096
E = 8
K = 2
A = S * K
TB = 512
P = A + E * TB
NB = P // TB
FFC = 1024
CH = 64
NW = 32


def _router_body(x_ref, wr_ref, probs_ref, ti_ref, tw_ref, pos_ref, be_ref):
    x = x_ref[...]
    wr = wr_ref[...]
    logits = lax.dot_general(
        x, wr, (((1,), (1,)), ((), ())), preferred_element_type=jnp.float32)
    m = jnp.max(logits, axis=1, keepdims=True)
    ex = jnp.exp(logits - m)
    probs = ex / jnp.sum(ex, axis=1, keepdims=True)
    probs_ref[...] = probs

    lane = lax.broadcasted_iota(jnp.int32, (S, E), 1)
    m0 = jnp.max(probs, axis=1, keepdims=True)
    i0 = jnp.min(jnp.where(probs == m0, lane, E), axis=1, keepdims=True)
    masked = jnp.where(lane == i0, -jnp.inf, probs)
    m1 = jnp.max(masked, axis=1, keepdims=True)
    i1 = jnp.min(jnp.where(masked == m1, lane, E), axis=1, keepdims=True)

    denom = m0 + m1
    ti_ref[...] = jnp.concatenate([i0, i1], axis=1)
    tw_ref[...] = jnp.concatenate([m0 / denom, m1 / denom], axis=1)

    h0 = (lane == i0).astype(jnp.bfloat16)
    h1 = (lane == i1).astype(jnp.bfloat16)
    mm = jnp.concatenate([h0, h1], axis=0)

    counts = jnp.sum(mm.astype(jnp.float32), axis=0, keepdims=True)
    nb = jnp.floor((counts + (TB - 1)) * (1.0 / TB))

    e_sub = lax.broadcasted_iota(jnp.int32, (E, E), 0)
    e_lane = lax.broadcasted_iota(jnp.int32, (E, E), 1)
    upper = (e_sub < e_lane).astype(jnp.bfloat16)
    off_row = lax.dot_general(
        nb.astype(jnp.bfloat16), upper, (((1,), (0,)), ((), ())),
        preferred_element_type=jnp.float32) * TB

    chunk = 512
    r_sub = lax.broadcasted_iota(jnp.int32, (chunk, chunk), 0)
    r_lane = lax.broadcasted_iota(jnp.int32, (chunk, chunk), 1)
    tril = (r_lane < r_sub).astype(jnp.bfloat16)
    carry = jnp.zeros((1, E), jnp.float32)
    pos_chunks = []
    for c in range(A // chunk):
        mc = mm[c * chunk:(c + 1) * chunk, :]
        cc = lax.dot_general(
            tril, mc, (((1,), (0,)), ((), ())),
            preferred_element_type=jnp.float32) + carry
        mcf = mc.astype(jnp.float32)
        posc = jnp.sum((cc + off_row) * mcf, axis=1, keepdims=True)
        pos_chunks.append(posc)
        carry = carry + jnp.sum(mcf, axis=0, keepdims=True)
    pos_ref[...] = jnp.concatenate(pos_chunks, axis=0).astype(jnp.int32)

    counts_col = lax.dot_general(
        mm, jnp.ones((A, 1), jnp.bfloat16), (((0,), (0,)), ((), ())),
        preferred_element_type=jnp.float32)
    nb_col = jnp.floor((counts_col + (TB - 1)) * (1.0 / TB))
    lower = (e_lane < e_sub).astype(jnp.bfloat16)
    boff_col = lax.dot_general(
        lower, nb_col, (((1,), (0,)), ((), ())),
        preferred_element_type=jnp.float32)
    b_lane = lax.broadcasted_iota(jnp.int32, (E, 128), 1)
    bmask = (boff_col <= b_lane.astype(jnp.float32)).astype(jnp.float32)
    be = jnp.sum(bmask, axis=0, keepdims=True) - 1.0
    used = jnp.sum(nb)
    one_lane = lax.broadcasted_iota(jnp.int32, (1, 128), 1)
    be_ref[...] = jnp.where(one_lane == NB, used, be).astype(jnp.int32)


def _router(x2d, wr):
    return pl.pallas_call(
        _router_body,
        out_shape=(
            jax.ShapeDtypeStruct((S, E), jnp.float32),
            jax.ShapeDtypeStruct((S, K), jnp.int32),
            jax.ShapeDtypeStruct((S, K), jnp.float32),
            jax.ShapeDtypeStruct((A, 1), jnp.int32),
            jax.ShapeDtypeStruct((1, 128), jnp.int32),
        ),
    )(x2d, wr)


def _sc_dispatch(x2d, tid, pos):
    mesh = plsc.VectorSubcoreMesh(core_axis_name="c", subcore_axis_name="s")

    @functools.partial(
        pl.kernel, mesh=mesh,
        out_type=jax.ShapeDtypeStruct((P, D), jnp.float32),
        scratch_types=[
            pltpu.VMEM((CH,), jnp.int32),
            pltpu.VMEM((CH,), jnp.int32),
            pltpu.VMEM((CH, D), jnp.float32),
            pltpu.SemaphoreType.DMA,
        ],
    )
    def disp(x_hbm, tid_hbm, pos_hbm, xg_hbm, tid_v, pos_v, rows_v, sem):
        wid = lax.axis_index("s") * 2 + lax.axis_index("c")
        for j in range(A // (NW * CH)):
            base = wid * (A // NW) + j * CH
            pltpu.sync_copy(tid_hbm.at[pl.ds(base, CH)], tid_v)
            pltpu.sync_copy(pos_hbm.at[pl.ds(base, CH)], pos_v)
            pltpu.async_copy(x_hbm.at[tid_v], rows_v, sem).wait()
            pltpu.async_copy(rows_v, xg_hbm.at[pos_v], sem).wait()

    return disp(x2d, tid, pos)


def _sc_collect(og, pos):
    mesh = plsc.VectorSubcoreMesh(core_axis_name="c", subcore_axis_name="s")

    @functools.partial(
        pl.kernel, mesh=mesh,
        out_type=jax.ShapeDtypeStruct((A, D), jnp.float32),
        scratch_types=[
            pltpu.VMEM((CH,), jnp.int32),
            pltpu.VMEM((CH, D), jnp.float32),
            pltpu.SemaphoreType.DMA,
        ],
    )
    def coll(og_hbm, pos_hbm, gall_hbm, pos_v, rows_v, sem):
        wid = lax.axis_index("s") * 2 + lax.axis_index("c")
        for j in range(A // (NW * CH)):
            base = wid * (A // NW) + j * CH
            pltpu.sync_copy(pos_hbm.at[pl.ds(base, CH)], pos_v)
            pltpu.async_copy(og_hbm.at[pos_v], rows_v, sem).wait()
            pltpu.sync_copy(rows_v, gall_hbm.at[pl.ds(base, CH)])

    return coll(og, pos)


FFH = FF // 2


def _ffn_acc(xg_ref, w1_ref, w2_ref):
    xb = xg_ref[...].astype(jnp.bfloat16)
    acc = jnp.zeros((TB, D), jnp.float32)
    for fc in range(FFH // FFC):
        w1c = w1_ref[0, fc * FFC:(fc + 1) * FFC, :].astype(jnp.bfloat16)
        h = lax.dot_general(
            xb, w1c, (((1,), (1,)), ((), ())),
            preferred_element_type=jnp.float32)
        h = h * jax.nn.sigmoid(h)
        w2c = w2_ref[0, :, fc * FFC:(fc + 1) * FFC].astype(jnp.bfloat16)
        acc = acc + lax.dot_general(
            h.astype(jnp.bfloat16), w2c, (((1,), (1,)), ((), ())),
            preferred_element_type=jnp.float32)
    return acc


def _ffn_body_a(sc_ref, xg_ref, w1_ref, w2_ref, o_ref):
    @pl.when(pl.program_id(0) < sc_ref[NB])
    def _():
        o_ref[...] = _ffn_acc(xg_ref, w1_ref, w2_ref).astype(jnp.bfloat16)


def _ffn_body_b(sc_ref, xg_ref, w1_ref, w2_ref, prev_ref, o_ref):
    @pl.when(pl.program_id(0) < sc_ref[NB])
    def _():
        o_ref[...] = (prev_ref[...].astype(jnp.float32)
                      + _ffn_acc(xg_ref, w1_ref, w2_ref))


def _grouped_ffn(scalars, xg, w1, w2):
    spec_a = pltpu.PrefetchScalarGridSpec(
        num_scalar_prefetch=1,
        grid=(NB,),
        in_specs=[
            pl.BlockSpec((TB, D), lambda b, sc: (b, 0)),
            pl.BlockSpec((1, FFH, D), lambda b, sc: (sc[b], 0, 0)),
            pl.BlockSpec((1, D, FFH), lambda b, sc: (sc[b], 0, 0)),
        ],
        out_specs=pl.BlockSpec((TB, D), lambda b, sc: (b, 0)),
    )
    og0 = pl.pallas_call(
        _ffn_body_a,
        grid_spec=spec_a,
        out_shape=jax.ShapeDtypeStruct((P, D), jnp.bfloat16),
    )(scalars, xg, w1, w2)
    spec_b = pltpu.PrefetchScalarGridSpec(
        num_scalar_prefetch=1,
        grid=(NB,),
        in_specs=[
            pl.BlockSpec((TB, D), lambda b, sc: (b, 0)),
            pl.BlockSpec((1, FFH, D), lambda b, sc: (sc[b], 1, 0)),
            pl.BlockSpec((1, D, FFH), lambda b, sc: (sc[b], 0, 1)),
            pl.BlockSpec((TB, D), lambda b, sc: (b, 0)),
        ],
        out_specs=pl.BlockSpec((TB, D), lambda b, sc: (b, 0)),
    )
    return pl.pallas_call(
        _ffn_body_b,
        grid_spec=spec_b,
        out_shape=jax.ShapeDtypeStruct((P, D), jnp.float32),
    )(scalars, xg, w1, w2, og0)


def _combine_body(g_ref, tw_ref, o_ref):
    g0 = g_ref[0:S, :]
    g1 = g_ref[S:A, :]
    o_ref[...] = tw_ref[:, 0:1] * g0 + tw_ref[:, 1:2] * g1


def _combine(g_all, tw):
    return pl.pallas_call(
        _combine_body,
        out_shape=jax.ShapeDtypeStruct((S, D), jnp.float32),
    )(g_all, tw)


@jax.jit
def kernel(x, Wr, w1, w2):
    x2d = x.reshape(S, D)
    probs, ti, tw, posf, bev = _router(x2d, Wr)
    pos = posf.reshape(A)
    tid = jnp.tile(jnp.arange(S, dtype=jnp.int32), K)
    xg = _sc_dispatch(x2d, tid, pos)
    scalars = bev[0, :NB + 1]
    og = _grouped_ffn(scalars, xg, w1, w2)
    g_all = _sc_collect(og, pos)
    out = _combine(g_all, tw)
    return (out.reshape(1, S, D), probs.reshape(1, S, E),
            ti.reshape(1, S, 2), tw.reshape(1, S, 2))


# speedup vs baseline: 1.9719x; 1.0021x over previous
import functools

import jax
import jax.numpy as jnp
from jax import lax
from jax.experimental import pallas as pl
from jax.experimental.pallas import tpu as pltpu
from jax.experimental.pallas import tpu_sc as plsc

S = 2048
D = 1024
FF = 4096
E = 8
K = 2
A = S * K
TB = 512
P = A + E * TB
NB = P // TB
FFC = 1024
CH = 64
NW = 32


def _router_body(x_ref, wr_ref, probs_ref, ti_ref, tw_ref, pos_ref, be_ref):
    x = x_ref[...]
    wr = wr_ref[...]
    logits = lax.dot_general(
        x, wr, (((1,), (1,)), ((), ())), preferred_element_type=jnp.float32)
    m = jnp.max(logits, axis=1, keepdims=True)
    ex = jnp.exp(logits - m)
    probs = ex / jnp.sum(ex, axis=1, keepdims=True)
    probs_ref[...] = probs

    lane = lax.broadcasted_iota(jnp.int32, (S, E), 1)
    m0 = jnp.max(probs, axis=1, keepdims=True)
    i0 = jnp.min(jnp.where(probs == m0, lane, E), axis=1, keepdims=True)
    masked = jnp.where(lane == i0, -jnp.inf, probs)
    m1 = jnp.max(masked, axis=1, keepdims=True)
    i1 = jnp.min(jnp.where(masked == m1, lane, E), axis=1, keepdims=True)

    denom = m0 + m1
    ti_ref[...] = jnp.concatenate([i0, i1], axis=1)
    tw_ref[...] = jnp.concatenate([m0 / denom, m1 / denom], axis=1)

    h0 = (lane == i0).astype(jnp.bfloat16)
    h1 = (lane == i1).astype(jnp.bfloat16)
    mm = jnp.concatenate([h0, h1], axis=0)

    counts = jnp.sum(mm.astype(jnp.float32), axis=0, keepdims=True)
    nb = jnp.floor((counts + (TB - 1)) * (1.0 / TB))

    e_sub = lax.broadcasted_iota(jnp.int32, (E, E), 0)
    e_lane = lax.broadcasted_iota(jnp.int32, (E, E), 1)
    upper = (e_sub < e_lane).astype(jnp.bfloat16)
    off_row = lax.dot_general(
        nb.astype(jnp.bfloat16), upper, (((1,), (0,)), ((), ())),
        preferred_element_type=jnp.float32) * TB

    chunk = 512
    r_sub = lax.broadcasted_iota(jnp.int32, (chunk, chunk), 0)
    r_lane = lax.broadcasted_iota(jnp.int32, (chunk, chunk), 1)
    tril = (r_lane < r_sub).astype(jnp.bfloat16)
    carry = jnp.zeros((1, E), jnp.float32)
    pos_chunks = []
    for c in range(A // chunk):
        mc = mm[c * chunk:(c + 1) * chunk, :]
        cc = lax.dot_general(
            tril, mc, (((1,), (0,)), ((), ())),
            preferred_element_type=jnp.float32) + carry
        mcf = mc.astype(jnp.float32)
        posc = jnp.sum((cc + off_row) * mcf, axis=1, keepdims=True)
        pos_chunks.append(posc)
        carry = carry + jnp.sum(mcf, axis=0, keepdims=True)
    pos_ref[...] = jnp.concatenate(pos_chunks, axis=0).astype(jnp.int32)

    counts_col = lax.dot_general(
        mm, jnp.ones((A, 1), jnp.bfloat16), (((0,), (0,)), ((), ())),
        preferred_element_type=jnp.float32)
    nb_col = jnp.floor((counts_col + (TB - 1)) * (1.0 / TB))
    lower = (e_lane < e_sub).astype(jnp.bfloat16)
    boff_col = lax.dot_general(
        lower, nb_col, (((1,), (0,)), ((), ())),
        preferred_element_type=jnp.float32)
    b_lane = lax.broadcasted_iota(jnp.int32, (E, 128), 1)
    bmask = (boff_col <= b_lane.astype(jnp.float32)).astype(jnp.float32)
    be = jnp.sum(bmask, axis=0, keepdims=True) - 1.0
    used = jnp.sum(nb)
    one_lane = lax.broadcasted_iota(jnp.int32, (1, 128), 1)
    be_ref[...] = jnp.where(one_lane == NB, used, be).astype(jnp.int32)


def _router(x2d, wr):
    return pl.pallas_call(
        _router_body,
        out_shape=(
            jax.ShapeDtypeStruct((S, E), jnp.float32),
            jax.ShapeDtypeStruct((S, K), jnp.int32),
            jax.ShapeDtypeStruct((S, K), jnp.float32),
            jax.ShapeDtypeStruct((A, 1), jnp.int32),
            jax.ShapeDtypeStruct((1, 128), jnp.int32),
        ),
    )(x2d, wr)


def _sc_dispatch(x2d, tid, pos):
    mesh = plsc.VectorSubcoreMesh(core_axis_name="c", subcore_axis_name="s")

    @functools.partial(
        pl.kernel, mesh=mesh,
        out_type=jax.ShapeDtypeStruct((P, D), jnp.float32),
        scratch_types=[
            pltpu.VMEM((CH,), jnp.int32),
            pltpu.VMEM((CH,), jnp.int32),
            pltpu.VMEM((CH, D), jnp.float32),
            pltpu.SemaphoreType.DMA,
        ],
    )
    def disp(x_hbm, tid_hbm, pos_hbm, xg_hbm, tid_v, pos_v, rows_v, sem):
        wid = lax.axis_index("s") * 2 + lax.axis_index("c")
        for j in range(A // (NW * CH)):
            base = wid * (A // NW) + j * CH
            pltpu.sync_copy(tid_hbm.at[pl.ds(base, CH)], tid_v)
            pltpu.sync_copy(pos_hbm.at[pl.ds(base, CH)], pos_v)
            pltpu.async_copy(x_hbm.at[tid_v], rows_v, sem).wait()
            pltpu.async_copy(rows_v, xg_hbm.at[pos_v], sem).wait()

    return disp(x2d, tid, pos)


def _sc_collect(og, pos):
    mesh = plsc.VectorSubcoreMesh(core_axis_name="c", subcore_axis_name="s")

    @functools.partial(
        pl.kernel, mesh=mesh,
        out_type=jax.ShapeDtypeStruct((A, D), jnp.float32),
        scratch_types=[
            pltpu.VMEM((CH,), jnp.int32),
            pltpu.VMEM((CH, D), jnp.float32),
            pltpu.SemaphoreType.DMA,
        ],
    )
    def coll(og_hbm, pos_hbm, gall_hbm, pos_v, rows_v, sem):
        wid = lax.axis_index("s") * 2 + lax.axis_index("c")
        for j in range(A // (NW * CH)):
            base = wid * (A // NW) + j * CH
            pltpu.sync_copy(pos_hbm.at[pl.ds(base, CH)], pos_v)
            pltpu.async_copy(og_hbm.at[pos_v], rows_v, sem).wait()
            pltpu.sync_copy(rows_v, gall_hbm.at[pl.ds(base, CH)])

    return coll(og, pos)


FFH = FF // 2


def _ffn_acc(xg_ref, w1_ref, w2_ref):
    xb = xg_ref[...].astype(jnp.bfloat16)
    acc = jnp.zeros((TB, D), jnp.float32)
    for fc in range(FFH // FFC):
        w1c = w1_ref[0, fc * FFC:(fc + 1) * FFC, :].astype(jnp.bfloat16)
        h = lax.dot_general(
            xb, w1c, (((1,), (1,)), ((), ())),
            preferred_element_type=jnp.float32)
        h = h * jax.nn.sigmoid(h)
        w2c = w2_ref[0, :, fc * FFC:(fc + 1) * FFC].astype(jnp.bfloat16)
        acc = acc + lax.dot_general(
            h.astype(jnp.bfloat16), w2c, (((1,), (1,)), ((), ())),
            preferred_element_type=jnp.float32)
    return acc


def _ffn_body_a(sc_ref, xg_ref, w1_ref, w2_ref, o_ref):
    @pl.when(pl.program_id(0) < sc_ref[NB])
    def _():
        o_ref[...] = _ffn_acc(xg_ref, w1_ref, w2_ref).astype(jnp.bfloat16)


def _ffn_body_b(sc_ref, xg_ref, w1_ref, w2_ref, prev_ref, o_ref):
    @pl.when(pl.program_id(0) < sc_ref[NB])
    def _():
        o_ref[...] = (prev_ref[...].astype(jnp.float32)
                      + _ffn_acc(xg_ref, w1_ref, w2_ref))


def _grouped_ffn(scalars, xg, w1, w2):
    spec_a = pltpu.PrefetchScalarGridSpec(
        num_scalar_prefetch=1,
        grid=(NB,),
        in_specs=[
            pl.BlockSpec((TB, D), lambda b, sc: (b, 0)),
            pl.BlockSpec((1, FFH, D), lambda b, sc: (sc[b], 0, 0)),
            pl.BlockSpec((1, D, FFH), lambda b, sc: (sc[b], 0, 0)),
        ],
        out_specs=pl.BlockSpec((TB, D), lambda b, sc: (b, 0)),
    )
    og0 = pl.pallas_call(
        _ffn_body_a,
        grid_spec=spec_a,
        out_shape=jax.ShapeDtypeStruct((P, D), jnp.bfloat16),
    )(scalars, xg, w1, w2)
    spec_b = pltpu.PrefetchScalarGridSpec(
        num_scalar_prefetch=1,
        grid=(NB,),
        in_specs=[
            pl.BlockSpec((TB, D), lambda b, sc: (b, 0)),
            pl.BlockSpec((1, FFH, D), lambda b, sc: (sc[b], 1, 0)),
            pl.BlockSpec((1, D, FFH), lambda b, sc: (sc[b], 0, 1)),
            pl.BlockSpec((TB, D), lambda b, sc: (b, 0)),
        ],
        out_specs=pl.BlockSpec((TB, D), lambda b, sc: (b, 0)),
    )
    return pl.pallas_call(
        _ffn_body_b,
        grid_spec=spec_b,
        out_shape=jax.ShapeDtypeStruct((P, D), jnp.float32),
    )(scalars, xg, w1, w2, og0)


def _combine_body(g_ref, tw_ref, o_ref):
    g0 = g_ref[0:S, :].astype(jnp.float32)
    g1 = g_ref[S:A, :].astype(jnp.float32)
    o_ref[...] = tw_ref[:, 0:1] * g0 + tw_ref[:, 1:2] * g1


def _combine(g_all, tw):
    return pl.pallas_call(
        _combine_body,
        out_shape=jax.ShapeDtypeStruct((S, D), jnp.float32),
    )(g_all, tw)


@jax.jit
def kernel(x, Wr, w1, w2):
    x2d = x.reshape(S, D)
    probs, ti, tw, posf, bev = _router(x2d, Wr)
    pos = posf.reshape(A)
    tid = jnp.tile(jnp.arange(S, dtype=jnp.int32), K)
    xg = _sc_dispatch(x2d, tid, pos)
    scalars = bev[0, :NB + 1]
    og = _grouped_ffn(scalars, xg, w1, w2)
    g_all = _sc_collect(og, pos)
    out = _combine(g_all, tw)
    return (out.reshape(1, S, D), probs.reshape(1, S, E),
            ti.reshape(1, S, 2), tw.reshape(1, S, 2))
